# baseline (speedup 1.0000x reference)
import functools

import jax
import jax.numpy as jnp
from jax import lax
from jax.experimental import pallas as pl
from jax.experimental.pallas import tpu as pltpu
from jax.experimental.pallas import tpu_sc as plsc

B = 2
S = 2048
H = 2048
I = 7168
E = 8
PRIME = 5099
N = B * S

T = 256
P = N + E * T
NT = P // T
P2 = P + N
NT2 = P2 // T
TI = 512
NI = I // TI

TIS = 512
NIS = I // TIS

TA = 256
T2 = 1024
TC = 512

NW = 32


def _mix_recept_body(x_ref, xprev_ref, mk_ref, mr_ref, wr_ref, xk_ref, r_ref):
    t = pl.program_id(0)
    x = x_ref[...]
    prev_last = xprev_ref[TA - 1:TA, :]
    first_of_batch = (t % (S // TA)) == 0
    prev_last = jnp.where(first_of_batch, jnp.zeros_like(prev_last), prev_last)
    shifted = jnp.concatenate([prev_last, x[:TA - 1, :]], axis=0)
    delta = shifted - x
    xk_ref[...] = x + delta * mk_ref[...]
    xr = x + delta * mr_ref[...]
    z = lax.dot_general(xr, wr_ref[...], (((1,), (1,)), ((), ())),
                        preferred_element_type=jnp.float32)
    r_ref[...] = jax.nn.sigmoid(z)


def _ffn_body(te_ref, tv_ref, xg_ref, wk_ref, wv_ref, wks_ref, wvs_ref,
              out_ref):
    t = pl.program_id(0)
    i = pl.program_id(1)

    def accumulate(y):
        @pl.when(i == 0)
        def _():
            out_ref[...] = y

        @pl.when(i > 0)
        def _():
            out_ref[...] += y

    @pl.when(tv_ref[t] == 1)
    def _():
        a = lax.dot_general(xg_ref[...], wk_ref[0], (((1,), (1,)), ((), ())),
                            preferred_element_type=jnp.float32)
        hh = jnp.square(jnp.maximum(a, 0.0))
        accumulate(lax.dot_general(hh, wv_ref[0], (((1,), (1,)), ((), ())),
                                   preferred_element_type=jnp.float32))

    @pl.when(tv_ref[t] == 2)
    def _():
        a = lax.dot_general(xg_ref[...], wks_ref[...], (((1,), (1,)), ((), ())),
                            preferred_element_type=jnp.float32)
        hh = jnp.square(jnp.maximum(a, 0.0))
        accumulate(lax.dot_general(hh, wvs_ref[...], (((1,), (1,)), ((), ())),
                                   preferred_element_type=jnp.float32))


def _combine_body(r_ref, vsh_ref, ye_ref, out_ref):
    out_ref[...] = r_ref[...] * (vsh_ref[...] + ye_ref[...])


@functools.lru_cache(maxsize=None)
def _make_sc_row_gather(n_rows_out, chunk):
    rows_per_w = n_rows_out // NW
    n_chunks = rows_per_w // chunk

    @functools.partial(
        pl.kernel,
        mesh=plsc.VectorSubcoreMesh(core_axis_name="c", subcore_axis_name="s"),
        out_type=jax.ShapeDtypeStruct((n_rows_out, H), jnp.float32),
        scratch_types=[
            pltpu.VMEM((rows_per_w,), jnp.int32),
            pltpu.VMEM((chunk, H), jnp.float32),
            pltpu.VMEM((chunk, H), jnp.float32),
            pltpu.SemaphoreType.DMA,
            pltpu.SemaphoreType.DMA,
        ],
    )
    def gather(table_hbm, idx_hbm, out_hbm, idx_v, rows0, rows1, sem0, sem1):
        wid = lax.axis_index("s") * 2 + lax.axis_index("c")
        base = wid * rows_per_w
        pltpu.sync_copy(idx_hbm.at[pl.ds(base, rows_per_w)], idx_v)
        bufs = (rows0, rows1)
        sems = (sem0, sem1)
        copies = [None, None]
        copies[0] = pltpu.async_copy(
            table_hbm.at[idx_v.at[pl.ds(0, chunk)]], rows0, sem0)
        for c in range(n_chunks):
            if c + 1 < n_chunks:
                copies[(c + 1) % 2] = pltpu.async_copy(
                    table_hbm.at[idx_v.at[pl.ds((c + 1) * chunk, chunk)]],
                    bufs[(c + 1) % 2], sems[(c + 1) % 2])
            copies[c % 2].wait()
            pltpu.sync_copy(bufs[c % 2],
                            out_hbm.at[pl.ds(base + c * chunk, chunk)])

    return gather


def kernel(hidden, input_ids, time_maa_k, time_maa_r, W_r, Wk_shared,
           Wv_shared, Wk_experts, Wv_experts):
    x = hidden.reshape(N, H)
    mk = time_maa_k.reshape(1, H)
    mr = time_maa_r.reshape(1, H)

    flat_ids = input_ids.reshape(N).astype(jnp.int32)
    routes = (flat_ids * PRIME) % E
    onehot = (routes[:, None] == jnp.arange(E, dtype=jnp.int32)[None, :])
    ranks_all = jnp.cumsum(onehot.astype(jnp.int32), axis=0)
    rank = jnp.take_along_axis(ranks_all, routes[:, None], axis=1)[:, 0] - 1
    counts = ranks_all[-1]
    aligned = ((counts + T - 1) // T) * T
    starts = jnp.concatenate([jnp.zeros(1, jnp.int32),
                              jnp.cumsum(aligned)[:-1].astype(jnp.int32)])
    ends = starts + aligned
    pos = starts[routes] + rank
    gather_idx = jnp.zeros((P,), jnp.int32).at[pos].set(
        jnp.arange(N, dtype=jnp.int32))
    gather_idx = jnp.concatenate([gather_idx,
                                  jnp.arange(N, dtype=jnp.int32)])
    tile_starts = jnp.arange(NT, dtype=jnp.int32) * T
    tile_expert = jnp.sum(
        (tile_starts[:, None] >= ends[None, :]).astype(jnp.int32), axis=1)
    tile_valid = (tile_starts < ends[E - 1]).astype(jnp.int32)
    tile_expert = jnp.minimum(tile_expert, E - 1)
    tile_expert = jnp.concatenate(
        [tile_expert, jnp.full((NT2 - NT,), E - 1, jnp.int32)])
    tile_valid = jnp.concatenate(
        [tile_valid, jnp.full((NT2 - NT,), 2, jnp.int32)])

    xk, r = pl.pallas_call(
        _mix_recept_body,
        grid=(N // TA,),
        in_specs=[
            pl.BlockSpec((TA, H), lambda t: (t, 0)),
            pl.BlockSpec((TA, H), lambda t: (jnp.maximum(t - 1, 0), 0)),
            pl.BlockSpec((1, H), lambda t: (0, 0)),
            pl.BlockSpec((1, H), lambda t: (0, 0)),
            pl.BlockSpec((H, H), lambda t: (0, 0)),
        ],
        out_specs=[
            pl.BlockSpec((TA, H), lambda t: (t, 0)),
            pl.BlockSpec((TA, H), lambda t: (t, 0)),
        ],
        out_shape=[
            jax.ShapeDtypeStruct((N, H), jnp.float32),
            jax.ShapeDtypeStruct((N, H), jnp.float32),
        ],
        compiler_params=pltpu.CompilerParams(
            dimension_semantics=("arbitrary",)),
    )(x, x, mk, mr, W_r)

    xg = _make_sc_row_gather(P2, 16)(xk, gather_idx)

    yg = pl.pallas_call(
        _ffn_body,
        grid_spec=pltpu.PrefetchScalarGridSpec(
            num_scalar_prefetch=2,
            grid=(NT2, NI),
            in_specs=[
                pl.BlockSpec((T, H), lambda t, i, te, tv: (t, 0)),
                pl.BlockSpec((1, TI, H), lambda t, i, te, tv: (te[t], i, 0)),
                pl.BlockSpec((1, H, TI), lambda t, i, te, tv: (te[t], 0, i)),
                pl.BlockSpec((TI, H), lambda t, i, te, tv: (i, 0)),
                pl.BlockSpec((H, TI), lambda t, i, te, tv: (0, i)),
            ],
            out_specs=pl.BlockSpec((T, H), lambda t, i, te, tv: (t, 0)),
        ),
        out_shape=jax.ShapeDtypeStruct((P2, H), jnp.float32),
        compiler_params=pltpu.CompilerParams(
            dimension_semantics=("arbitrary", "arbitrary")),
    )(tile_expert, tile_valid, xg, Wk_experts, Wv_experts,
      Wk_shared, Wv_shared)

    ye = _make_sc_row_gather(N, 16)(yg, pos)

    out = pl.pallas_call(
        _combine_body,
        grid=(N // TC,),
        in_specs=[
            pl.BlockSpec((TC, H), lambda t: (t, 0)),
            pl.BlockSpec((TC, H), lambda t: (P // TC + t, 0)),
            pl.BlockSpec((TC, H), lambda t: (t, 0)),
        ],
        out_specs=pl.BlockSpec((TC, H), lambda t: (t, 0)),
        out_shape=jax.ShapeDtypeStruct((N, H), jnp.float32),
        compiler_params=pltpu.CompilerParams(
            dimension_semantics=("arbitrary",)),
    )(r, yg, ye)

    return out.reshape(B, S, H)


# speedup vs baseline: 1.9559x; 1.9559x over previous
import functools

import jax
import jax.numpy as jnp
from jax import lax
from jax.experimental import pallas as pl
from jax.experimental.pallas import tpu as pltpu
from jax.experimental.pallas import tpu_sc as plsc

B = 2
S = 2048
H = 2048
I = 7168
E = 8
PRIME = 5099
N = B * S

T = 256
P = N + E * T
NT = P // T
TI = 896
NI = I // TI
TIC = 896
NIC = I // TIC

TIS = 512
NIS = I // TIS

TA = 256
T2 = 1024
TC = 512

NW = 32


def _mix_recept_body(x_ref, xprev_ref, mk_ref, mr_ref, wr_ref, xk_ref, r_ref):
    t = pl.program_id(0)
    x = x_ref[...]
    prev_last = xprev_ref[TA - 1:TA, :]
    first_of_batch = (t % (S // TA)) == 0
    prev_last = jnp.where(first_of_batch, jnp.zeros_like(prev_last), prev_last)
    shifted = jnp.concatenate([prev_last, x[:TA - 1, :]], axis=0)
    delta = shifted - x
    xk_ref[...] = x + delta * mk_ref[...]
    xr = x + delta * mr_ref[...]
    z = lax.dot_general(xr, wr_ref[...], (((1,), (1,)), ((), ())),
                        preferred_element_type=jnp.float32)
    r_ref[...] = jax.nn.sigmoid(z)


def _shared_ffn_body(xk_ref, wk_ref, wv_ref, out_ref):
    i = pl.program_id(1)
    a = lax.dot_general(xk_ref[...], wk_ref[...], (((1,), (1,)), ((), ())),
                        preferred_element_type=jnp.float32)
    hh = jnp.square(jnp.maximum(a, 0.0))
    y = lax.dot_general(hh, wv_ref[...], (((1,), (1,)), ((), ())),
                        preferred_element_type=jnp.float32)

    @pl.when(i == 0)
    def _():
        out_ref[...] = y

    @pl.when(i > 0)
    def _():
        out_ref[...] += y


def _wcast_body(wk_ref, wv_ref, wkb_ref, wvb_ref):
    wkb_ref[...] = wk_ref[...].astype(jnp.bfloat16)
    wvb_ref[...] = wv_ref[...].astype(jnp.bfloat16)


def _expert_ffn_body(te_ref, tv_ref, xg_ref, wk_ref, wv_ref, vdummy_ref,
                     out_ref):
    t = pl.program_id(0)
    i = pl.program_id(1)

    @pl.when(tv_ref[t] > 0)
    def _():
        a = lax.dot_general(xg_ref[...].astype(jnp.bfloat16), wk_ref[0],
                            (((1,), (1,)), ((), ())),
                            preferred_element_type=jnp.float32)
        hh = jnp.square(jnp.maximum(a, 0.0)).astype(jnp.bfloat16)
        y = lax.dot_general(hh, wv_ref[0], (((1,), (1,)), ((), ())),
                            preferred_element_type=jnp.float32)

        @pl.when(i == 0)
        def _():
            out_ref[...] = y

        @pl.when(i > 0)
        def _():
            out_ref[...] += y


def _combine_body(r_ref, vsh_ref, ye_ref, out_ref):
    out_ref[...] = r_ref[...] * (vsh_ref[...] + ye_ref[...])


@functools.lru_cache(maxsize=None)
def _make_sc_row_gather(n_rows_out, chunk):
    rows_per_w = n_rows_out // NW
    n_chunks = rows_per_w // chunk

    @functools.partial(
        pl.kernel,
        mesh=plsc.VectorSubcoreMesh(core_axis_name="c", subcore_axis_name="s"),
        out_type=jax.ShapeDtypeStruct((n_rows_out, H), jnp.float32),
        scratch_types=[
            pltpu.VMEM((rows_per_w,), jnp.int32),
            pltpu.VMEM((chunk, H), jnp.float32),
            pltpu.VMEM((chunk, H), jnp.float32),
            pltpu.SemaphoreType.DMA,
            pltpu.SemaphoreType.DMA,
        ],
    )
    def gather(table_hbm, idx_hbm, out_hbm, idx_v, rows0, rows1, sem0, sem1):
        wid = lax.axis_index("s") * 2 + lax.axis_index("c")
        base = wid * rows_per_w
        pltpu.sync_copy(idx_hbm.at[pl.ds(base, rows_per_w)], idx_v)
        bufs = (rows0, rows1)
        sems = (sem0, sem1)
        copies = [None, None]
        copies[0] = pltpu.async_copy(
            table_hbm.at[idx_v.at[pl.ds(0, chunk)]], rows0, sem0)
        for c in range(n_chunks):
            if c + 1 < n_chunks:
                copies[(c + 1) % 2] = pltpu.async_copy(
                    table_hbm.at[idx_v.at[pl.ds((c + 1) * chunk, chunk)]],
                    bufs[(c + 1) % 2], sems[(c + 1) % 2])
            copies[c % 2].wait()
            pltpu.sync_copy(bufs[c % 2],
                            out_hbm.at[pl.ds(base + c * chunk, chunk)])

    return gather


def kernel(hidden, input_ids, time_maa_k, time_maa_r, W_r, Wk_shared,
           Wv_shared, Wk_experts, Wv_experts):
    x = hidden.reshape(N, H)
    mk = time_maa_k.reshape(1, H)
    mr = time_maa_r.reshape(1, H)

    flat_ids = input_ids.reshape(N).astype(jnp.int32)
    routes = (flat_ids * PRIME) % E
    onehot = (routes[:, None] == jnp.arange(E, dtype=jnp.int32)[None, :])
    ranks_all = jnp.cumsum(onehot.astype(jnp.int32), axis=0)
    rank = jnp.take_along_axis(ranks_all, routes[:, None], axis=1)[:, 0] - 1
    counts = ranks_all[-1]
    aligned = ((counts + T - 1) // T) * T
    starts = jnp.concatenate([jnp.zeros(1, jnp.int32),
                              jnp.cumsum(aligned)[:-1].astype(jnp.int32)])
    ends = starts + aligned
    pos = starts[routes] + rank
    gather_idx = jnp.zeros((P,), jnp.int32).at[pos].set(
        jnp.arange(N, dtype=jnp.int32))
    tile_starts = jnp.arange(NT, dtype=jnp.int32) * T
    tile_expert = jnp.sum(
        (tile_starts[:, None] >= ends[None, :]).astype(jnp.int32), axis=1)
    tile_valid = (tile_starts < ends[E - 1]).astype(jnp.int32)
    tile_expert = jnp.minimum(tile_expert, E - 1)

    xk, r = pl.pallas_call(
        _mix_recept_body,
        grid=(N // TA,),
        in_specs=[
            pl.BlockSpec((TA, H), lambda t: (t, 0)),
            pl.BlockSpec((TA, H), lambda t: (jnp.maximum(t - 1, 0), 0)),
            pl.BlockSpec((1, H), lambda t: (0, 0)),
            pl.BlockSpec((1, H), lambda t: (0, 0)),
            pl.BlockSpec((H, H), lambda t: (0, 0)),
        ],
        out_specs=[
            pl.BlockSpec((TA, H), lambda t: (t, 0)),
            pl.BlockSpec((TA, H), lambda t: (t, 0)),
        ],
        out_shape=[
            jax.ShapeDtypeStruct((N, H), jnp.float32),
            jax.ShapeDtypeStruct((N, H), jnp.float32),
        ],
        compiler_params=pltpu.CompilerParams(
            dimension_semantics=("arbitrary",)),
    )(x, x, mk, mr, W_r)

    xg = _make_sc_row_gather(P, 16)(xk, gather_idx)

    v_sh = pl.pallas_call(
        _shared_ffn_body,
        grid=(N // T2, NIS),
        in_specs=[
            pl.BlockSpec((T2, H), lambda t, i: (t, 0)),
            pl.BlockSpec((TIS, H), lambda t, i: (i, 0)),
            pl.BlockSpec((H, TIS), lambda t, i: (0, i)),
        ],
        out_specs=pl.BlockSpec((T2, H), lambda t, i: (t, 0)),
        out_shape=jax.ShapeDtypeStruct((N, H), jnp.float32),
        compiler_params=pltpu.CompilerParams(
            dimension_semantics=("arbitrary", "arbitrary")),
    )(xk, Wk_shared, Wv_shared)

    wk_e_b, wv_e_b = pl.pallas_call(
        _wcast_body,
        grid=(E, NIC),
        in_specs=[
            pl.BlockSpec((1, TIC, H), lambda e, i: (e, i, 0)),
            pl.BlockSpec((1, H, TIC), lambda e, i: (e, 0, i)),
        ],
        out_specs=[
            pl.BlockSpec((1, TIC, H), lambda e, i: (e, i, 0)),
            pl.BlockSpec((1, H, TIC), lambda e, i: (e, 0, i)),
        ],
        out_shape=[
            jax.ShapeDtypeStruct((E, I, H), jnp.bfloat16),
            jax.ShapeDtypeStruct((E, H, I), jnp.bfloat16),
        ],
        compiler_params=pltpu.CompilerParams(
            dimension_semantics=("arbitrary", "arbitrary")),
    )(Wk_experts, Wv_experts)

    yg = pl.pallas_call(
        _expert_ffn_body,
        grid_spec=pltpu.PrefetchScalarGridSpec(
            num_scalar_prefetch=2,
            grid=(NT, NI),
            in_specs=[
                pl.BlockSpec((T, H), lambda t, i, te, tv: (t, 0)),
                pl.BlockSpec((1, TI, H), lambda t, i, te, tv: (te[t], i, 0)),
                pl.BlockSpec((1, H, TI), lambda t, i, te, tv: (te[t], 0, i)),
                pl.BlockSpec((8, 128), lambda t, i, te, tv: (0, 0)),
            ],
            out_specs=pl.BlockSpec((T, H), lambda t, i, te, tv: (t, 0)),
        ),
        out_shape=jax.ShapeDtypeStruct((P, H), jnp.float32),
        compiler_params=pltpu.CompilerParams(
            dimension_semantics=("arbitrary", "arbitrary")),
    )(tile_expert, tile_valid, xg, wk_e_b, wv_e_b, v_sh)

    ye = _make_sc_row_gather(N, 16)(yg, pos)

    out = pl.pallas_call(
        _combine_body,
        grid=(N // TC,),
        in_specs=[
            pl.BlockSpec((TC, H), lambda t: (t, 0)),
            pl.BlockSpec((TC, H), lambda t: (t, 0)),
            pl.BlockSpec((TC, H), lambda t: (t, 0)),
        ],
        out_specs=pl.BlockSpec((TC, H), lambda t: (t, 0)),
        out_shape=jax.ShapeDtypeStruct((N, H), jnp.float32),
        compiler_params=pltpu.CompilerParams(
            dimension_semantics=("arbitrary",)),
    )(r, v_sh, ye)

    return out.reshape(B, S, H)


# speedup vs baseline: 2.1991x; 1.1243x over previous
import functools

import jax
import jax.numpy as jnp
from jax import lax
from jax.experimental import pallas as pl
from jax.experimental.pallas import tpu as pltpu
from jax.experimental.pallas import tpu_sc as plsc

B = 2
S = 2048
H = 2048
I = 7168
E = 8
PRIME = 5099
N = B * S

T = 512
P = N + E * T
NT = P // T
TI = 896
NI = I // TI

TIS = 512
NIS = I // TIS

TA = 256
T2 = 1024
TC = 512

NW = 32


def _mix_recept_body(x_ref, xprev_ref, mk_ref, mr_ref, wr_ref, xk_ref, r_ref):
    t = pl.program_id(0)
    x = x_ref[...]
    prev_last = xprev_ref[TA - 1:TA, :]
    first_of_batch = (t % (S // TA)) == 0
    prev_last = jnp.where(first_of_batch, jnp.zeros_like(prev_last), prev_last)
    shifted = jnp.concatenate([prev_last, x[:TA - 1, :]], axis=0)
    delta = shifted - x
    xk_ref[...] = x + delta * mk_ref[...]
    xr = x + delta * mr_ref[...]
    z = lax.dot_general(xr, wr_ref[...], (((1,), (1,)), ((), ())),
                        preferred_element_type=jnp.float32)
    r_ref[...] = jax.nn.sigmoid(z)


def _shared_ffn_body(xk_ref, wk_ref, wv_ref, out_ref):
    i = pl.program_id(1)
    a = lax.dot_general(xk_ref[...], wk_ref[...], (((1,), (1,)), ((), ())),
                        preferred_element_type=jnp.float32)
    hh = jnp.square(jnp.maximum(a, 0.0))
    y = lax.dot_general(hh, wv_ref[...], (((1,), (1,)), ((), ())),
                        preferred_element_type=jnp.float32)

    @pl.when(i == 0)
    def _():
        out_ref[...] = y

    @pl.when(i > 0)
    def _():
        out_ref[...] += y


def _expert_ffn_body(te_ref, tv_ref, xg_ref, wk_ref, wv_ref, vdummy_ref,
                     out_ref):
    t = pl.program_id(0)
    i = pl.program_id(1)

    @pl.when(tv_ref[t] > 0)
    def _():
        a = lax.dot_general(xg_ref[...], wk_ref[0], (((1,), (1,)), ((), ())),
                            preferred_element_type=jnp.float32)
        hh = jnp.square(jnp.maximum(a, 0.0))
        y = lax.dot_general(hh, wv_ref[0], (((1,), (1,)), ((), ())),
                            preferred_element_type=jnp.float32)

        @pl.when(i == 0)
        def _():
            out_ref[...] = y

        @pl.when(i > 0)
        def _():
            out_ref[...] += y


def _combine_body(r_ref, vsh_ref, ye_ref, out_ref):
    out_ref[...] = r_ref[...] * (vsh_ref[...] + ye_ref[...])


@functools.lru_cache(maxsize=None)
def _make_sc_row_gather(n_rows_out, chunk):
    rows_per_w = n_rows_out // NW
    n_chunks = rows_per_w // chunk

    @functools.partial(
        pl.kernel,
        mesh=plsc.VectorSubcoreMesh(core_axis_name="c", subcore_axis_name="s"),
        out_type=jax.ShapeDtypeStruct((n_rows_out, H), jnp.float32),
        scratch_types=[
            pltpu.VMEM((rows_per_w,), jnp.int32),
            pltpu.VMEM((chunk, H), jnp.float32),
            pltpu.VMEM((chunk, H), jnp.float32),
            pltpu.SemaphoreType.DMA,
            pltpu.SemaphoreType.DMA,
        ],
    )
    def gather(table_hbm, idx_hbm, out_hbm, idx_v, rows0, rows1, sem0, sem1):
        wid = lax.axis_index("s") * 2 + lax.axis_index("c")
        base = wid * rows_per_w
        pltpu.sync_copy(idx_hbm.at[pl.ds(base, rows_per_w)], idx_v)
        bufs = (rows0, rows1)
        sems = (sem0, sem1)
        copies = [None, None]
        copies[0] = pltpu.async_copy(
            table_hbm.at[idx_v.at[pl.ds(0, chunk)]], rows0, sem0)
        for c in range(n_chunks):
            if c + 1 < n_chunks:
                copies[(c + 1) % 2] = pltpu.async_copy(
                    table_hbm.at[idx_v.at[pl.ds((c + 1) * chunk, chunk)]],
                    bufs[(c + 1) % 2], sems[(c + 1) % 2])
            copies[c % 2].wait()
            pltpu.sync_copy(bufs[c % 2],
                            out_hbm.at[pl.ds(base + c * chunk, chunk)])

    return gather


def kernel(hidden, input_ids, time_maa_k, time_maa_r, W_r, Wk_shared,
           Wv_shared, Wk_experts, Wv_experts):
    x = hidden.reshape(N, H)
    mk = time_maa_k.reshape(1, H)
    mr = time_maa_r.reshape(1, H)

    flat_ids = input_ids.reshape(N).astype(jnp.int32)
    routes = (flat_ids * PRIME) % E
    onehot = (routes[:, None] == jnp.arange(E, dtype=jnp.int32)[None, :])
    ranks_all = jnp.cumsum(onehot.astype(jnp.int32), axis=0)
    rank = jnp.take_along_axis(ranks_all, routes[:, None], axis=1)[:, 0] - 1
    counts = ranks_all[-1]
    aligned = ((counts + T - 1) // T) * T
    starts = jnp.concatenate([jnp.zeros(1, jnp.int32),
                              jnp.cumsum(aligned)[:-1].astype(jnp.int32)])
    ends = starts + aligned
    pos = starts[routes] + rank
    gather_idx = jnp.zeros((P,), jnp.int32).at[pos].set(
        jnp.arange(N, dtype=jnp.int32))
    tile_starts = jnp.arange(NT, dtype=jnp.int32) * T
    tile_expert = jnp.sum(
        (tile_starts[:, None] >= ends[None, :]).astype(jnp.int32), axis=1)
    tile_valid = (tile_starts < ends[E - 1]).astype(jnp.int32)
    tile_expert = jnp.minimum(tile_expert, E - 1)

    xk, r = pl.pallas_call(
        _mix_recept_body,
        grid=(N // TA,),
        in_specs=[
            pl.BlockSpec((TA, H), lambda t: (t, 0)),
            pl.BlockSpec((TA, H), lambda t: (jnp.maximum(t - 1, 0), 0)),
            pl.BlockSpec((1, H), lambda t: (0, 0)),
            pl.BlockSpec((1, H), lambda t: (0, 0)),
            pl.BlockSpec((H, H), lambda t: (0, 0)),
        ],
        out_specs=[
            pl.BlockSpec((TA, H), lambda t: (t, 0)),
            pl.BlockSpec((TA, H), lambda t: (t, 0)),
        ],
        out_shape=[
            jax.ShapeDtypeStruct((N, H), jnp.float32),
            jax.ShapeDtypeStruct((N, H), jnp.float32),
        ],
        compiler_params=pltpu.CompilerParams(
            dimension_semantics=("arbitrary",)),
    )(x, x, mk, mr, W_r)

    xg = _make_sc_row_gather(P, 16)(xk, gather_idx)

    v_sh = pl.pallas_call(
        _shared_ffn_body,
        grid=(N // T2, NIS),
        in_specs=[
            pl.BlockSpec((T2, H), lambda t, i: (t, 0)),
            pl.BlockSpec((TIS, H), lambda t, i: (i, 0)),
            pl.BlockSpec((H, TIS), lambda t, i: (0, i)),
        ],
        out_specs=pl.BlockSpec((T2, H), lambda t, i: (t, 0)),
        out_shape=jax.ShapeDtypeStruct((N, H), jnp.float32),
        compiler_params=pltpu.CompilerParams(
            dimension_semantics=("arbitrary", "arbitrary")),
    )(xk, Wk_shared, Wv_shared)

    yg = pl.pallas_call(
        _expert_ffn_body,
        grid_spec=pltpu.PrefetchScalarGridSpec(
            num_scalar_prefetch=2,
            grid=(NT, NI),
            in_specs=[
                pl.BlockSpec((T, H), lambda t, i, te, tv: (t, 0)),
                pl.BlockSpec((1, TI, H), lambda t, i, te, tv: (te[t], i, 0)),
                pl.BlockSpec((1, H, TI), lambda t, i, te, tv: (te[t], 0, i)),
                pl.BlockSpec((8, 128), lambda t, i, te, tv: (0, 0)),
            ],
            out_specs=pl.BlockSpec((T, H), lambda t, i, te, tv: (t, 0)),
        ),
        out_shape=jax.ShapeDtypeStruct((P, H), jnp.float32),
        compiler_params=pltpu.CompilerParams(
            dimension_semantics=("arbitrary", "arbitrary")),
    )(tile_expert, tile_valid, xg, Wk_experts, Wv_experts, v_sh)

    ye = _make_sc_row_gather(N, 16)(yg, pos)

    out = pl.pallas_call(
        _combine_body,
        grid=(N // TC,),
        in_specs=[
            pl.BlockSpec((TC, H), lambda t: (t, 0)),
            pl.BlockSpec((TC, H), lambda t: (t, 0)),
            pl.BlockSpec((TC, H), lambda t: (t, 0)),
        ],
        out_specs=pl.BlockSpec((TC, H), lambda t: (t, 0)),
        out_shape=jax.ShapeDtypeStruct((N, H), jnp.float32),
        compiler_params=pltpu.CompilerParams(
            dimension_semantics=("arbitrary",)),
    )(r, v_sh, ye)

    return out.reshape(B, S, H)
